# baseline (speedup 1.0000x reference)
import jax
import jax.numpy as jnp
from jax import lax
from jax.experimental import pallas as pl
from jax.experimental.pallas import tpu as pltpu
from jax.experimental.pallas import tpu_sc as plsc

N = 10000
E = 320000
D = 128

NUM_CORES = 2
NUM_SUBCORES = 16
CHUNK = 128
NUM_CHUNKS = E // CHUNK
T = 160
PAD_CHUNKS = T * NUM_SUBCORES
ACC_ROWS = N + 8
PHASES = 4
PCH = T // PHASES
ROWS_PER_TILE = 624
ROWS_TAIL = N - ROWS_PER_TILE * NUM_SUBCORES

_MESH = plsc.VectorSubcoreMesh(core_axis_name="c", subcore_axis_name="s",
                               num_cores=NUM_CORES,
                               num_subcores=NUM_SUBCORES)


def _for_row_chunks(s, fn):
    base = s * ROWS_PER_TILE
    off = 0
    for n in (CHUNK, CHUNK, CHUNK, CHUNK, ROWS_PER_TILE - 4 * CHUNK):
        fn(base + off, n)
        off += n

    @pl.when(s == NUM_SUBCORES - 1)
    def _():
        fn(ROWS_PER_TILE * NUM_SUBCORES, ROWS_TAIL)


NBUF = 2


def _sc_agg_body(x_hbm, src_hbm, dst_hbm, zrows_hbm, agg_hbm,
                 acc, srcb, dstb, rows0, rows1, g0, g1, t0, t1):
    c = lax.axis_index("c")
    s = lax.axis_index("s")
    rows = (rows0, rows1)
    gsem = (g0, g1)
    tsem = (t0, t1)

    pltpu.sync_copy(zrows_hbm, rows0)
    _for_row_chunks(
        s, lambda base, n: pltpu.sync_copy(rows0.at[pl.ds(0, n)],
                                           acc.at[pl.ds(base, n)]))
    plsc.subcore_barrier()

    x_dir = x_hbm.at[c]

    def gather(j, i):
        pltpu.async_copy(x_dir.at[srcb.at[j]], rows[i], gsem[i])

    def gwait(j, i):
        pltpu.make_async_copy(x_dir.at[srcb.at[j]], rows[i], gsem[i]).wait()

    def scat(j, i):
        pltpu.async_copy(rows[i], acc.at[dstb.at[j]], tsem[i], add=True)

    def swait(j, i):
        pltpu.make_async_copy(rows[i], acc.at[dstb.at[j]], tsem[i]).wait()

    def phase(p, carry):
        pltpu.sync_copy(src_hbm.at[c, s, pl.ds(p * PCH, PCH)], srcb)
        pltpu.sync_copy(dst_hbm.at[c, s, pl.ds(p * PCH, PCH)], dstb)
        for i in range(NBUF):
            gather(i, i)

        def body(m, carry2):
            j = NBUF * m
            for i in range(NBUF):
                gwait(j + i, i)
                scat(j + i, i)
            for i in range(NBUF):
                swait(j + i, i)
                gather(j + NBUF + i, i)
            return carry2

        lax.fori_loop(0, PCH // NBUF - 1, body, 0)
        j = PCH - NBUF
        for i in range(NBUF):
            gwait(j + i, i)
            scat(j + i, i)
        for i in range(NBUF):
            swait(j + i, i)
        return carry

    lax.fori_loop(0, PHASES, phase, 0)
    plsc.subcore_barrier()

    def out_chunk(base, n):
        pltpu.sync_copy(acc.at[pl.ds(base, n)], rows0.at[pl.ds(0, n)])
        pltpu.sync_copy(rows0.at[pl.ds(0, n)],
                        agg_hbm.at[c, pl.ds(base, n)])

    _for_row_chunks(s, out_chunk)


_sc_agg = pl.kernel(
    _sc_agg_body,
    out_type=jax.ShapeDtypeStruct((NUM_CORES, N, D), jnp.float32),
    mesh=_MESH,
    scratch_types=[
        pltpu.VMEM_SHARED((ACC_ROWS, D), jnp.float32),
        pltpu.VMEM((PCH, CHUNK), jnp.int32),
        pltpu.VMEM((PCH, CHUNK), jnp.int32),
        pltpu.VMEM((CHUNK, D), jnp.float32),
        pltpu.VMEM((CHUNK, D), jnp.float32),
        pltpu.SemaphoreType.DMA,
        pltpu.SemaphoreType.DMA,
        pltpu.SemaphoreType.DMA,
        pltpu.SemaphoreType.DMA,
    ],
)


def _sc_deg_body(dst_hbm, oz_hbm, deg_hbm, dacc, dstb, buf, sem):
    c = lax.axis_index("c")
    s = lax.axis_index("s")

    pltpu.sync_copy(oz_hbm.at[1], buf)
    _for_row_chunks(
        s, lambda base, n: pltpu.sync_copy(buf.at[pl.ds(0, n)],
                                           dacc.at[pl.ds(base, n)]))
    pltpu.sync_copy(oz_hbm.at[0], buf)
    plsc.subcore_barrier()

    BATCH = 8

    def phase(p, carry):
        pltpu.sync_copy(dst_hbm.at[c, s, pl.ds(p * PCH, PCH)], dstb)

        def body(m, carry2):
            for i in range(BATCH):
                j = m * BATCH + i
                pltpu.async_copy(buf, dacc.at[dstb.at[j]], sem, add=True)
            for i in range(BATCH):
                j = m * BATCH + i
                pltpu.make_async_copy(buf, dacc.at[dstb.at[j]], sem).wait()
            return carry2

        lax.fori_loop(0, PCH // BATCH, body, 0)
        return carry

    lax.fori_loop(0, PHASES, phase, 0)
    plsc.subcore_barrier()

    def out_chunk(base, n):
        pltpu.sync_copy(dacc.at[pl.ds(base, n)], buf.at[pl.ds(0, n)])
        pltpu.sync_copy(buf.at[pl.ds(0, n)],
                        deg_hbm.at[c, pl.ds(base, n)])

    _for_row_chunks(s, out_chunk)


_sc_deg = pl.kernel(
    _sc_deg_body,
    out_type=jax.ShapeDtypeStruct((NUM_CORES, N, D), jnp.float32),
    mesh=_MESH,
    scratch_types=[
        pltpu.VMEM_SHARED((ACC_ROWS, D), jnp.float32),
        pltpu.VMEM((PCH, CHUNK), jnp.int32),
        pltpu.VMEM((CHUNK, D), jnp.float32),
        pltpu.SemaphoreType.DMA,
    ],
)


def _tc_linear_body(agg_ref, deg_ref, x_ref, wl_ref, b_ref, wr_ref, o_ref):
    a = agg_ref[0]
    d = deg_ref[0][:, :1]
    mean = a * (1.0 / jnp.maximum(d, 1.0))
    o = lax.dot_general(mean, wl_ref[0], (((1,), (1,)), ((), ())),
                        preferred_element_type=jnp.float32)
    o = o + lax.dot_general(x_ref[0], wr_ref[0], (((1,), (1,)), ((), ())),
                            preferred_element_type=jnp.float32)
    o_ref[0] = o + b_ref[0]


def _tc_linear(agg, deg, x_cat, wl, b, wr, agg_other, x_other):
    B = 1000
    rb = N // B

    def agg_map(k, r):
        return ((1 - k) if agg_other else k, r, 0)

    def x_map(k, r):
        return ((1 - k) if x_other else k, r, 0)

    return pl.pallas_call(
        _tc_linear_body,
        grid=(2, rb),
        in_specs=[
            pl.BlockSpec((1, B, D), agg_map),
            pl.BlockSpec((1, B, D), agg_map),
            pl.BlockSpec((1, B, D), x_map),
            pl.BlockSpec((1, D, D), lambda k, r: (k, 0, 0)),
            pl.BlockSpec((1, 1, D), lambda k, r: (k, 0, 0)),
            pl.BlockSpec((1, D, D), lambda k, r: (k, 0, 0)),
        ],
        out_specs=pl.BlockSpec((1, B, D), lambda k, r: (k, r, 0)),
        out_shape=jax.ShapeDtypeStruct((2, N, D), jnp.float32),
    )(agg, deg, x_cat, wl, b, wr)


def _prep_idx(e, pad_val):
    a = e.astype(jnp.int32).reshape(NUM_CHUNKS, CHUNK)
    pad = jnp.full((PAD_CHUNKS - NUM_CHUNKS, CHUNK), pad_val, jnp.int32)
    a = jnp.concatenate([a, pad], 0).reshape(T, NUM_SUBCORES, CHUNK)
    return a.transpose(1, 0, 2)


def kernel(x_s, x_t, s2t_edge_index, t2s_edge_index,
           Wl1s, bl1s, Wr1s, Wl1t, bl1t, Wr1t,
           Wl2s, bl2s, Wr2s, Wl2t, bl2t, Wr2t):
    x_cat = jnp.stack([x_t, x_s])
    src_cat = jnp.stack([_prep_idx(t2s_edge_index[0], 0),
                         _prep_idx(s2t_edge_index[0], 0)])
    dst_cat = jnp.stack([_prep_idx(t2s_edge_index[1], N),
                         _prep_idx(s2t_edge_index[1], N)])

    zrows = jnp.zeros((CHUNK, D), jnp.float32)
    oz = jnp.stack([jnp.ones((CHUNK, D), jnp.float32), zrows])

    deg = _sc_deg(dst_cat, oz)
    agg1 = _sc_agg(x_cat, src_cat, dst_cat, zrows)

    wl1 = jnp.stack([Wl1t, Wl1s])
    b1 = jnp.stack([bl1t, bl1s]).reshape(2, 1, D)
    wr1 = jnp.stack([Wr1t, Wr1s])
    h = _tc_linear(agg1, deg, x_cat, wl1, b1, wr1, agg_other=True,
                   x_other=False)

    agg2 = _sc_agg(h, src_cat, dst_cat, zrows)

    wl2 = jnp.stack([Wl2s, Wl2t])
    b2 = jnp.stack([bl2s, bl2t]).reshape(2, 1, D)
    wr2 = jnp.stack([Wr2s, Wr2t])
    z = _tc_linear(agg2, deg, h, wl2, b2, wr2, agg_other=False, x_other=True)

    return (z[0], z[1])


# speedup vs baseline: 1.0929x; 1.0929x over previous
import jax
import jax.numpy as jnp
from jax import lax
from jax.experimental import pallas as pl
from jax.experimental.pallas import tpu as pltpu
from jax.experimental.pallas import tpu_sc as plsc

N = 10000
E = 320000
D = 128

NUM_CORES = 2
NUM_SUBCORES = 16
CHUNK = 128
NUM_CHUNKS = E // CHUNK
T = 160
PAD_CHUNKS = T * NUM_SUBCORES
ACC_ROWS = N + 8
PHASES = 4
PCH = T // PHASES
U = 64
UP = PCH * CHUNK // U
ROWS_PER_TILE = 624
ROWS_TAIL = N - ROWS_PER_TILE * NUM_SUBCORES

_MESH = plsc.VectorSubcoreMesh(core_axis_name="c", subcore_axis_name="s",
                               num_cores=NUM_CORES,
                               num_subcores=NUM_SUBCORES)


def _for_row_chunks(s, fn):
    base = s * ROWS_PER_TILE
    off = 0
    for n in (U,) * 9 + (ROWS_PER_TILE - 9 * U,):
        fn(base + off, n)
        off += n

    @pl.when(s == NUM_SUBCORES - 1)
    def _():
        fn(ROWS_PER_TILE * NUM_SUBCORES, ROWS_TAIL)


NBUF = 4


def _sc_agg_body(x_hbm, src_hbm, dst_hbm, zrows_hbm, agg_hbm,
                 acc, srcb, dstb, rows0, rows1, rows2, rows3,
                 g0, g1, g2, g3, t0, t1, t2, t3):
    c = lax.axis_index("c")
    s = lax.axis_index("s")
    rows = (rows0, rows1, rows2, rows3)
    gsem = (g0, g1, g2, g3)
    tsem = (t0, t1, t2, t3)

    pltpu.sync_copy(zrows_hbm, rows0)
    _for_row_chunks(
        s, lambda base, n: pltpu.sync_copy(rows0.at[pl.ds(0, n)],
                                           acc.at[pl.ds(base, n)]))
    plsc.subcore_barrier()

    x_dir = x_hbm.at[c]

    def idx(b, j, h):
        return b.at[j, pl.ds(h * U, U)]

    def gather(j, h, i):
        pltpu.async_copy(x_dir.at[idx(srcb, j, h)], rows[i], gsem[i])

    def gwait(j, h, i):
        pltpu.make_async_copy(x_dir.at[idx(srcb, j, h)], rows[i],
                              gsem[i]).wait()

    def scat(j, h, i):
        pltpu.async_copy(rows[i], acc.at[idx(dstb, j, h)], tsem[i],
                         add=True)

    def swait(j, h, i):
        pltpu.make_async_copy(rows[i], acc.at[idx(dstb, j, h)],
                              tsem[i]).wait()

    def phase(p, carry):
        pltpu.sync_copy(src_hbm.at[c, s, pl.ds(p * PCH, PCH)], srcb)
        pltpu.sync_copy(dst_hbm.at[c, s, pl.ds(p * PCH, PCH)], dstb)
        for i in range(NBUF):
            gather(i // 2, i % 2, i)

        def body(g, carry2):
            j0 = 4 * g
            for i in range(NBUF):
                j, h = j0 + i // 2, i % 2
                gwait(j, h, i)
                scat(j, h, i)
            for i in range(NBUF):
                j, h = j0 + i // 2, i % 2
                swait(j, h, i)
                gather(j + 2, h, i)
            for i in range(NBUF):
                j, h = j0 + 2 + i // 2, i % 2
                gwait(j, h, i)
                scat(j, h, i)
            for i in range(NBUF):
                j, h = j0 + 2 + i // 2, i % 2
                swait(j, h, i)
                gather(j + 2, h, i)
            return carry2

        lax.fori_loop(0, UP // 8 - 1, body, 0)
        j0 = PCH - 4
        for i in range(NBUF):
            j, h = j0 + i // 2, i % 2
            gwait(j, h, i)
            scat(j, h, i)
        for i in range(NBUF):
            j, h = j0 + i // 2, i % 2
            swait(j, h, i)
            gather(j + 2, h, i)
        for i in range(NBUF):
            j, h = j0 + 2 + i // 2, i % 2
            gwait(j, h, i)
            scat(j, h, i)
        for i in range(NBUF):
            j, h = j0 + 2 + i // 2, i % 2
            swait(j, h, i)
        return carry

    lax.fori_loop(0, PHASES, phase, 0)
    plsc.subcore_barrier()

    def out_chunk(base, n):
        pltpu.sync_copy(acc.at[pl.ds(base, n)], rows0.at[pl.ds(0, n)])
        pltpu.sync_copy(rows0.at[pl.ds(0, n)],
                        agg_hbm.at[c, pl.ds(base, n)])

    _for_row_chunks(s, out_chunk)


_sc_agg = pl.kernel(
    _sc_agg_body,
    out_type=jax.ShapeDtypeStruct((NUM_CORES, N, D), jnp.float32),
    mesh=_MESH,
    scratch_types=[
        pltpu.VMEM_SHARED((ACC_ROWS, D), jnp.float32),
        pltpu.VMEM((PCH, CHUNK), jnp.int32),
        pltpu.VMEM((PCH, CHUNK), jnp.int32),
        pltpu.VMEM((U, D), jnp.float32),
        pltpu.VMEM((U, D), jnp.float32),
        pltpu.VMEM((U, D), jnp.float32),
        pltpu.VMEM((U, D), jnp.float32),
        pltpu.SemaphoreType.DMA,
        pltpu.SemaphoreType.DMA,
        pltpu.SemaphoreType.DMA,
        pltpu.SemaphoreType.DMA,
        pltpu.SemaphoreType.DMA,
        pltpu.SemaphoreType.DMA,
        pltpu.SemaphoreType.DMA,
        pltpu.SemaphoreType.DMA,
    ],
)


def _sc_deg_body(dst_hbm, oz_hbm, deg_hbm, dacc, dstb, buf, sem):
    c = lax.axis_index("c")
    s = lax.axis_index("s")

    pltpu.sync_copy(oz_hbm.at[1], buf)
    _for_row_chunks(
        s, lambda base, n: pltpu.sync_copy(buf.at[pl.ds(0, n)],
                                           dacc.at[pl.ds(base, n)]))
    pltpu.sync_copy(oz_hbm.at[0], buf)
    plsc.subcore_barrier()

    BATCH = 8

    def phase(p, carry):
        pltpu.sync_copy(dst_hbm.at[c, s, pl.ds(p * PCH, PCH)], dstb)

        def body(m, carry2):
            for i in range(BATCH):
                j = m * BATCH + i
                pltpu.async_copy(buf, dacc.at[dstb.at[j]], sem, add=True)
            for i in range(BATCH):
                j = m * BATCH + i
                pltpu.make_async_copy(buf, dacc.at[dstb.at[j]], sem).wait()
            return carry2

        lax.fori_loop(0, PCH // BATCH, body, 0)
        return carry

    lax.fori_loop(0, PHASES, phase, 0)
    plsc.subcore_barrier()

    def out_chunk(base, n):
        pltpu.sync_copy(dacc.at[pl.ds(base, n)], buf.at[pl.ds(0, n)])
        pltpu.sync_copy(buf.at[pl.ds(0, n)],
                        deg_hbm.at[c, pl.ds(base, n)])

    _for_row_chunks(s, out_chunk)


_sc_deg = pl.kernel(
    _sc_deg_body,
    out_type=jax.ShapeDtypeStruct((NUM_CORES, N, D), jnp.float32),
    mesh=_MESH,
    scratch_types=[
        pltpu.VMEM_SHARED((ACC_ROWS, D), jnp.float32),
        pltpu.VMEM((PCH, CHUNK), jnp.int32),
        pltpu.VMEM((CHUNK, D), jnp.float32),
        pltpu.SemaphoreType.DMA,
    ],
)


def _tc_linear_body(agg_ref, deg_ref, x_ref, wl_ref, b_ref, wr_ref, o_ref):
    a = agg_ref[0]
    d = deg_ref[0][:, :1]
    mean = a * (1.0 / jnp.maximum(d, 1.0))
    o = lax.dot_general(mean, wl_ref[0], (((1,), (1,)), ((), ())),
                        preferred_element_type=jnp.float32)
    o = o + lax.dot_general(x_ref[0], wr_ref[0], (((1,), (1,)), ((), ())),
                            preferred_element_type=jnp.float32)
    o_ref[0] = o + b_ref[0]


def _tc_linear(agg, deg, x_cat, wl, b, wr, agg_other, x_other):
    B = 1000
    rb = N // B

    def agg_map(k, r):
        return ((1 - k) if agg_other else k, r, 0)

    def x_map(k, r):
        return ((1 - k) if x_other else k, r, 0)

    return pl.pallas_call(
        _tc_linear_body,
        grid=(2, rb),
        in_specs=[
            pl.BlockSpec((1, B, D), agg_map),
            pl.BlockSpec((1, B, D), agg_map),
            pl.BlockSpec((1, B, D), x_map),
            pl.BlockSpec((1, D, D), lambda k, r: (k, 0, 0)),
            pl.BlockSpec((1, 1, D), lambda k, r: (k, 0, 0)),
            pl.BlockSpec((1, D, D), lambda k, r: (k, 0, 0)),
        ],
        out_specs=pl.BlockSpec((1, B, D), lambda k, r: (k, r, 0)),
        out_shape=jax.ShapeDtypeStruct((2, N, D), jnp.float32),
    )(agg, deg, x_cat, wl, b, wr)


def _prep_idx(e, pad_val):
    a = e.astype(jnp.int32).reshape(NUM_CHUNKS, CHUNK)
    pad = jnp.full((PAD_CHUNKS - NUM_CHUNKS, CHUNK), pad_val, jnp.int32)
    a = jnp.concatenate([a, pad], 0).reshape(T, NUM_SUBCORES, CHUNK)
    return a.transpose(1, 0, 2)


def kernel(x_s, x_t, s2t_edge_index, t2s_edge_index,
           Wl1s, bl1s, Wr1s, Wl1t, bl1t, Wr1t,
           Wl2s, bl2s, Wr2s, Wl2t, bl2t, Wr2t):
    x_cat = jnp.stack([x_t, x_s])
    src_cat = jnp.stack([_prep_idx(t2s_edge_index[0], 0),
                         _prep_idx(s2t_edge_index[0], 0)])
    dst_cat = jnp.stack([_prep_idx(t2s_edge_index[1], N),
                         _prep_idx(s2t_edge_index[1], N)])

    zrows = jnp.zeros((U, D), jnp.float32)
    oz = jnp.stack([jnp.ones((CHUNK, D), jnp.float32),
                    jnp.zeros((CHUNK, D), jnp.float32)])

    deg = _sc_deg(dst_cat, oz)
    agg1 = _sc_agg(x_cat, src_cat, dst_cat, zrows)

    wl1 = jnp.stack([Wl1t, Wl1s])
    b1 = jnp.stack([bl1t, bl1s]).reshape(2, 1, D)
    wr1 = jnp.stack([Wr1t, Wr1s])
    h = _tc_linear(agg1, deg, x_cat, wl1, b1, wr1, agg_other=True,
                   x_other=False)

    agg2 = _sc_agg(h, src_cat, dst_cat, zrows)

    wl2 = jnp.stack([Wl2s, Wl2t])
    b2 = jnp.stack([bl2s, bl2t]).reshape(2, 1, D)
    wr2 = jnp.stack([Wr2s, Wr2t])
    z = _tc_linear(agg2, deg, h, wl2, b2, wr2, agg_other=False, x_other=True)

    return (z[0], z[1])


# speedup vs baseline: 1.1172x; 1.0223x over previous
import jax
import jax.numpy as jnp
from jax import lax
from jax.experimental import pallas as pl
from jax.experimental.pallas import tpu as pltpu
from jax.experimental.pallas import tpu_sc as plsc

N = 10000
E = 320000
D = 128

NUM_CORES = 2
NUM_SUBCORES = 16
CHUNK = 128
NUM_CHUNKS = E // CHUNK
T = 160
PAD_CHUNKS = T * NUM_SUBCORES
ACC_ROWS = N + 8
PHASES = 4
PCH = T // PHASES
ROWS_PER_TILE = 624
ROWS_TAIL = N - ROWS_PER_TILE * NUM_SUBCORES

_MESH = plsc.VectorSubcoreMesh(core_axis_name="c", subcore_axis_name="s",
                               num_cores=NUM_CORES,
                               num_subcores=NUM_SUBCORES)


def _for_row_chunks(s, fn):
    base = s * ROWS_PER_TILE
    off = 0
    for n in (CHUNK, CHUNK, CHUNK, CHUNK, ROWS_PER_TILE - 4 * CHUNK):
        fn(base + off, n)
        off += n

    @pl.when(s == NUM_SUBCORES - 1)
    def _():
        fn(ROWS_PER_TILE * NUM_SUBCORES, ROWS_TAIL)


def _sc_agg_body(x_hbm, src_hbm, dst_hbm, zrows_hbm, agg_hbm,
                 acc, srcb, dstb, rows0, rows1, g0, g1):
    c = lax.axis_index("c")
    s = lax.axis_index("s")

    pltpu.sync_copy(zrows_hbm, rows0)
    _for_row_chunks(
        s, lambda base, n: pltpu.sync_copy(rows0.at[pl.ds(0, n)],
                                           acc.at[pl.ds(base, n)]))
    plsc.subcore_barrier()

    x_dir = x_hbm.at[c]

    def gather(j, buf, sem):
        pltpu.async_copy(x_dir.at[srcb.at[j]], buf, sem)

    def gwait(j, buf, sem):
        pltpu.make_async_copy(x_dir.at[srcb.at[j]], buf, sem).wait()

    def scat(j, buf):
        pltpu.sync_copy(buf, acc.at[dstb.at[j]], add=True)

    def phase(p, carry):
        pltpu.sync_copy(src_hbm.at[c, s, pl.ds(p * PCH, PCH)], srcb)
        pltpu.sync_copy(dst_hbm.at[c, s, pl.ds(p * PCH, PCH)], dstb)
        gather(0, rows0, g0)
        gather(1, rows1, g1)

        def body(m, carry2):
            j = 2 * m
            gwait(j, rows0, g0)
            scat(j, rows0)
            gather(j + 2, rows0, g0)
            gwait(j + 1, rows1, g1)
            scat(j + 1, rows1)
            gather(j + 3, rows1, g1)
            return carry2

        lax.fori_loop(0, PCH // 2 - 1, body, 0)
        gwait(PCH - 2, rows0, g0)
        scat(PCH - 2, rows0)
        gwait(PCH - 1, rows1, g1)
        scat(PCH - 1, rows1)
        return carry

    lax.fori_loop(0, PHASES, phase, 0)
    plsc.subcore_barrier()

    def out_chunk(base, n):
        pltpu.sync_copy(acc.at[pl.ds(base, n)], rows0.at[pl.ds(0, n)])
        pltpu.sync_copy(rows0.at[pl.ds(0, n)],
                        agg_hbm.at[c, pl.ds(base, n)])

    _for_row_chunks(s, out_chunk)


_sc_agg = pl.kernel(
    _sc_agg_body,
    out_type=jax.ShapeDtypeStruct((NUM_CORES, N, D), jnp.float32),
    mesh=_MESH,
    scratch_types=[
        pltpu.VMEM_SHARED((ACC_ROWS, D), jnp.float32),
        pltpu.VMEM((PCH, CHUNK), jnp.int32),
        pltpu.VMEM((PCH, CHUNK), jnp.int32),
        pltpu.VMEM((CHUNK, D), jnp.float32),
        pltpu.VMEM((CHUNK, D), jnp.float32),
        pltpu.SemaphoreType.DMA,
        pltpu.SemaphoreType.DMA,
    ],
)


def _sc_deg_body(dst_hbm, oz_hbm, deg_hbm, dacc, dstb, buf, sem):
    c = lax.axis_index("c")
    s = lax.axis_index("s")

    pltpu.sync_copy(oz_hbm.at[1], buf)
    _for_row_chunks(
        s, lambda base, n: pltpu.sync_copy(buf.at[pl.ds(0, n)],
                                           dacc.at[pl.ds(base, n)]))
    pltpu.sync_copy(oz_hbm.at[0], buf)
    plsc.subcore_barrier()

    BATCH = 8

    def phase(p, carry):
        pltpu.sync_copy(dst_hbm.at[c, s, pl.ds(p * PCH, PCH)], dstb)

        def body(m, carry2):
            for i in range(BATCH):
                j = m * BATCH + i
                pltpu.async_copy(buf, dacc.at[dstb.at[j]], sem, add=True)
            for i in range(BATCH):
                j = m * BATCH + i
                pltpu.make_async_copy(buf, dacc.at[dstb.at[j]], sem).wait()
            return carry2

        lax.fori_loop(0, PCH // BATCH, body, 0)
        return carry

    lax.fori_loop(0, PHASES, phase, 0)
    plsc.subcore_barrier()

    def out_chunk(base, n):
        pltpu.sync_copy(dacc.at[pl.ds(base, n)], buf.at[pl.ds(0, n)])
        pltpu.sync_copy(buf.at[pl.ds(0, n)],
                        deg_hbm.at[c, pl.ds(base, n)])

    _for_row_chunks(s, out_chunk)


_sc_deg = pl.kernel(
    _sc_deg_body,
    out_type=jax.ShapeDtypeStruct((NUM_CORES, N, D), jnp.float32),
    mesh=_MESH,
    scratch_types=[
        pltpu.VMEM_SHARED((ACC_ROWS, D), jnp.float32),
        pltpu.VMEM((PCH, CHUNK), jnp.int32),
        pltpu.VMEM((CHUNK, D), jnp.float32),
        pltpu.SemaphoreType.DMA,
    ],
)


def _tc_linear_body(agg_ref, deg_ref, x_ref, wl_ref, b_ref, wr_ref, o_ref):
    a = agg_ref[0]
    d = deg_ref[0][:, :1]
    mean = a * (1.0 / jnp.maximum(d, 1.0))
    o = lax.dot_general(mean, wl_ref[0], (((1,), (1,)), ((), ())),
                        preferred_element_type=jnp.float32)
    o = o + lax.dot_general(x_ref[0], wr_ref[0], (((1,), (1,)), ((), ())),
                            preferred_element_type=jnp.float32)
    o_ref[0] = o + b_ref[0]


def _tc_linear(agg, deg, x_cat, wl, b, wr, agg_other, x_other):
    B = 1000
    rb = N // B

    def agg_map(k, r):
        return ((1 - k) if agg_other else k, r, 0)

    def x_map(k, r):
        return ((1 - k) if x_other else k, r, 0)

    return pl.pallas_call(
        _tc_linear_body,
        grid=(2, rb),
        in_specs=[
            pl.BlockSpec((1, B, D), agg_map),
            pl.BlockSpec((1, B, D), agg_map),
            pl.BlockSpec((1, B, D), x_map),
            pl.BlockSpec((1, D, D), lambda k, r: (k, 0, 0)),
            pl.BlockSpec((1, 1, D), lambda k, r: (k, 0, 0)),
            pl.BlockSpec((1, D, D), lambda k, r: (k, 0, 0)),
        ],
        out_specs=pl.BlockSpec((1, B, D), lambda k, r: (k, r, 0)),
        out_shape=jax.ShapeDtypeStruct((2, N, D), jnp.float32),
    )(agg, deg, x_cat, wl, b, wr)


def _prep_idx(e, pad_val):
    a = e.astype(jnp.int32).reshape(NUM_CHUNKS, CHUNK)
    pad = jnp.full((PAD_CHUNKS - NUM_CHUNKS, CHUNK), pad_val, jnp.int32)
    a = jnp.concatenate([a, pad], 0).reshape(T, NUM_SUBCORES, CHUNK)
    return a.transpose(1, 0, 2)


def kernel(x_s, x_t, s2t_edge_index, t2s_edge_index,
           Wl1s, bl1s, Wr1s, Wl1t, bl1t, Wr1t,
           Wl2s, bl2s, Wr2s, Wl2t, bl2t, Wr2t):
    x_cat = jnp.stack([x_t, x_s])
    src_cat = jnp.stack([_prep_idx(t2s_edge_index[0], 0),
                         _prep_idx(s2t_edge_index[0], 0)])
    dst_cat = jnp.stack([_prep_idx(t2s_edge_index[1], N),
                         _prep_idx(s2t_edge_index[1], N)])

    zrows = jnp.zeros((CHUNK, D), jnp.float32)
    oz = jnp.stack([jnp.ones((CHUNK, D), jnp.float32), zrows])

    deg = _sc_deg(dst_cat, oz)
    agg1 = _sc_agg(x_cat, src_cat, dst_cat, zrows)

    wl1 = jnp.stack([Wl1t, Wl1s])
    b1 = jnp.stack([bl1t, bl1s]).reshape(2, 1, D)
    wr1 = jnp.stack([Wr1t, Wr1s])
    h = _tc_linear(agg1, deg, x_cat, wl1, b1, wr1, agg_other=True,
                   x_other=False)

    agg2 = _sc_agg(h, src_cat, dst_cat, zrows)

    wl2 = jnp.stack([Wl2s, Wl2t])
    b2 = jnp.stack([bl2s, bl2t]).reshape(2, 1, D)
    wr2 = jnp.stack([Wr2s, Wr2t])
    z = _tc_linear(agg2, deg, h, wl2, b2, wr2, agg_other=False, x_other=True)

    return (z[0], z[1])
